# baseline (speedup 1.0000x reference)
import functools

import jax
import jax.numpy as jnp
from jax.experimental import pallas as pl
from jax.experimental.pallas import tpu as pltpu

_EPS = 1e-12


def _gate_kernel(x_ref, w_ref, t_ref, out_ref, cinv_ref):
    @pl.when(pl.program_id(0) == 0)
    def _():
        w0 = w_ref[...]
        csq = jnp.maximum(jnp.sum(w0 * w0, axis=0, keepdims=True), _EPS * _EPS)
        cinv_ref[...] = jnp.exp(t_ref[0]) * jax.lax.rsqrt(csq)

    x = x_ref[...]
    acc = jnp.dot(x.astype(jnp.bfloat16), w_ref[...].astype(jnp.bfloat16),
                  preferred_element_type=jnp.float32)
    ssq = jnp.maximum(jnp.sum(x * x, axis=1, keepdims=True), _EPS * _EPS)
    rinv = jax.lax.rsqrt(ssq)
    out_ref[...] = acc * rinv * cinv_ref[...]


@functools.partial(jax.jit, static_argnames=("block_m",))
def _gate(hidden_states, sim_matrix, temperature, block_m):
    m, k = hidden_states.shape
    _, n = sim_matrix.shape
    grid = (m // block_m,)
    return pl.pallas_call(
        _gate_kernel,
        grid=grid,
        in_specs=[
            pl.BlockSpec((block_m, k), lambda i: (i, 0)),
            pl.BlockSpec((k, n), lambda i: (0, 0)),
            pl.BlockSpec(memory_space=pltpu.SMEM),
        ],
        out_specs=pl.BlockSpec((block_m, n), lambda i: (i, 0)),
        out_shape=jax.ShapeDtypeStruct((m, n), jnp.float32),
        scratch_shapes=[pltpu.VMEM((1, n), jnp.float32)],
    )(hidden_states, sim_matrix, temperature)


def kernel(hidden_states, sim_matrix, temperature):
    return _gate(hidden_states, sim_matrix, temperature, block_m=2048)
